# baseline (speedup 1.0000x reference)
import functools

import jax
import jax.numpy as jnp
from jax import lax
from jax.experimental import pallas as pl
from jax.experimental.pallas import tpu as pltpu
from jax.experimental.pallas import tpu_sc as plsc

NC = 2
NS = 16
L = 16
D = 128
CHUNK = 128
NBUF = 4

f32 = jnp.float32
i32 = jnp.int32

_SC_PARAMS = dict(
    compiler_params=pltpu.CompilerParams(needs_layout_passes=False))


def _deg_body(npad, nchunks, dst_hbm, out_hbm, hist, dall, sem):
    del sem
    cid = lax.axis_index("c")
    sid = lax.axis_index("s")
    wid = cid * NS + sid
    zeros = jnp.zeros((L,), f32)
    ones = jnp.ones((L,), f32)

    w0 = wid * nchunks
    pltpu.sync_copy(dst_hbm.at[pl.ds(w0, nchunks)], dall)

    def zero_body(i, _):
        hist[pl.ds(pl.multiple_of(i * L, L), L)] = zeros
        return 0
    lax.fori_loop(0, npad // L, zero_body, 0)

    def chunk_body(c, _):
        for j in range(CHUNK // L):
            idx = dall[c, pl.ds(j * L, L)]
            plsc.addupdate_scatter(hist, [idx], ones)
        return 0
    lax.fori_loop(0, nchunks, chunk_body, 0)

    pltpu.sync_copy(hist, out_hbm.at[wid])


def _make_deg(npad, epad):
    nchunks = epad // (NC * NS * CHUNK)
    mesh = plsc.VectorSubcoreMesh(core_axis_name="c", subcore_axis_name="s")
    return pl.kernel(
        functools.partial(_deg_body, npad, nchunks),
        out_type=jax.ShapeDtypeStruct((NC * NS, npad), f32),
        mesh=mesh,
        scratch_types=[
            pltpu.VMEM((npad,), f32),
            pltpu.VMEM((nchunks, CHUNK), i32),
            pltpu.SemaphoreType.DMA,
        ],
        **_SC_PARAMS,
    )


def _msg_body(npad, nchunks, tab_hbm, src_hbm, dst_hbm, out_hbm,
              acc_sh, rows, sibuf, dibuf,
              gs0, gs1, ss0, ss1, is0, is1, is2, is3, wsem):
    gsems = (gs0, gs1)
    ssems = (ss0, ss1)
    isems = (is0, is1, is2, is3)
    cid = lax.axis_index("c")
    sid = lax.axis_index("s")
    wid = cid * NS + sid
    rows_pt = npad // NS
    r0 = sid * rows_pt
    zeros = jnp.zeros((L,), f32)

    def zrow(r, _):
        for cc in range(D // L):
            rows[0, r, pl.ds(cc * L, L)] = zeros
        return 0
    lax.fori_loop(0, CHUNK, zrow, 0)
    nz = rows_pt // CHUNK
    for k in range(nz):
        pltpu.async_copy(rows.at[0], acc_sh.at[pl.ds(r0 + k * CHUNK, CHUNK)],
                         wsem)
    for k in range(nz):
        pltpu.make_async_copy(rows.at[0],
                              acc_sh.at[pl.ds(r0, CHUNK)], wsem).wait()
    plsc.subcore_barrier()

    w0 = wid * nchunks

    def fire_idx(c, q):
        pltpu.async_copy(src_hbm.at[c + w0], sibuf.at[q], isems[q])
        pltpu.async_copy(dst_hbm.at[c + w0], dibuf.at[q], isems[q])

    def wait_idx(q):
        for _ in range(2):
            pltpu.make_async_copy(src_hbm.at[0], sibuf.at[q],
                                  isems[q]).wait()

    def fire_g(q, b):
        h = CHUNK // 2
        pltpu.async_copy(tab_hbm.at[sibuf.at[q, pl.ds(0, h)]],
                         rows.at[b, pl.ds(0, h)], gsems[b])
        pltpu.async_copy(tab_hbm.at[sibuf.at[q, pl.ds(h, h)]],
                         rows.at[b, pl.ds(h, h)], gsems[b])

    def wait_g(b):
        h = CHUNK // 2
        for _ in range(2):
            pltpu.make_async_copy(tab_hbm.at[sibuf.at[0, pl.ds(0, h)]],
                                  rows.at[b, pl.ds(0, h)], gsems[b]).wait()

    def fire_s(q, b):
        pltpu.async_copy(rows.at[b], acc_sh.at[dibuf.at[q]], ssems[b],
                         add=True)

    def wait_s(b):
        pltpu.make_async_copy(rows.at[b], acc_sh.at[dibuf.at[0]],
                              ssems[b]).wait()

    for q in range(4):
        fire_idx(q, q)
    for c in range(2):
        wait_idx(c)
        fire_g(c, c)

    def iter_body(k, _):
        for cc in range(4):
            c = k * 4 + cc
            b = cc % 2
            q = cc
            wait_g(b)
            fire_s(q, b)
            wait_s(b)

            @pl.when(c + 4 < nchunks)
            def _():
                fire_idx(c + 4, q)

            @pl.when(c + 2 < nchunks)
            def _():
                wait_idx((q + 2) % 4)
                fire_g((q + 2) % 4, b)
        return 0
    lax.fori_loop(0, nchunks // 4, iter_body, 0)
    plsc.subcore_barrier()

    for k in range(nz):
        b = k % 2
        if k >= 2:
            pltpu.make_async_copy(rows.at[0],
                                  out_hbm.at[cid, pl.ds(r0, CHUNK)],
                                  wsem).wait()
        pltpu.sync_copy(acc_sh.at[pl.ds(r0 + k * CHUNK, CHUNK)], rows.at[b])
        pltpu.async_copy(rows.at[b],
                         out_hbm.at[cid, pl.ds(r0 + k * CHUNK, CHUNK)], wsem)
    for k in range(min(nz, 2)):
        pltpu.make_async_copy(rows.at[0], out_hbm.at[cid, pl.ds(r0, CHUNK)],
                              wsem).wait()


def _make_msg(npad, epad):
    nchunks = epad // (NC * NS * CHUNK)
    mesh = plsc.VectorSubcoreMesh(core_axis_name="c", subcore_axis_name="s")
    return pl.kernel(
        functools.partial(_msg_body, npad, nchunks),
        out_type=jax.ShapeDtypeStruct((NC, npad, D), f32),
        mesh=mesh,
        scratch_types=(
            [
                pltpu.VMEM_SHARED((npad, D), f32),
                pltpu.VMEM((2, CHUNK, D), f32),
                pltpu.VMEM((4, CHUNK), i32),
                pltpu.VMEM((4, CHUNK), i32),
            ]
            + [pltpu.SemaphoreType.DMA] * 9
        ),
        **_SC_PARAMS,
    )


R = 1024


def _dis_from_hist(hb):
    deg = jnp.sum(hb[...], axis=0) + 1.0
    return lax.rsqrt(deg)[:, None]


def _tc1_body(xb, wb, hb, hob):
    h = jnp.dot(xb[...], wb[...], preferred_element_type=f32)
    hob[...] = h * _dis_from_hist(hb)


def _tc2_body(mb, hb, histb, wb, bb, ob):
    dis = _dis_from_hist(histb)
    m = mb[...]
    z = (m[0] + m[1] + hb[...]) * dis + bb[...]
    z = jnp.maximum(z, 0.0)
    ob[...] = jnp.dot(z, wb[...], preferred_element_type=f32) * dis


def _tc3_body(mb, hb, histb, bb, ob):
    dis = _dis_from_hist(histb)
    m = mb[...]
    ob[...] = (m[0] + m[1] + hb[...]) * dis + bb[...]


def _make_tc(npad):
    nb = npad // R
    row = pl.BlockSpec((R, D), lambda i: (i, 0))
    full_w = pl.BlockSpec((D, D), lambda i: (0, 0))
    bias = pl.BlockSpec((1, D), lambda i: (0, 0))
    msg = pl.BlockSpec((NC, R, D), lambda i: (0, i, 0))
    hist = pl.BlockSpec((NC * NS, R), lambda i: (0, i))

    tc1 = pl.pallas_call(
        _tc1_body,
        grid=(nb,),
        in_specs=[row, full_w, hist],
        out_specs=row,
        out_shape=jax.ShapeDtypeStruct((npad, D), f32),
    )
    tc2 = pl.pallas_call(
        _tc2_body,
        grid=(nb,),
        in_specs=[msg, row, hist, full_w, bias],
        out_specs=row,
        out_shape=jax.ShapeDtypeStruct((npad, D), f32),
    )
    tc3 = pl.pallas_call(
        _tc3_body,
        grid=(nb,),
        in_specs=[msg, row, hist, bias],
        out_specs=row,
        out_shape=jax.ShapeDtypeStruct((npad, D), f32),
    )
    return tc1, tc2, tc3


def kernel(x, edge_index, W1, b1, W2, b2):
    n, d = x.shape
    e = edge_index.shape[1]
    assert d == D
    nstep = NS * CHUNK
    npad = pl.cdiv(n + 1, nstep) * nstep
    estep = NC * NS * CHUNK * NBUF
    epad = pl.cdiv(e, estep) * estep
    nch = epad // CHUNK

    ei = edge_index.astype(i32)
    pad = n + jnp.arange(epad - e, dtype=i32) % (npad - n)
    src = jnp.concatenate([ei[0], pad]).reshape(nch, CHUNK)
    dst = jnp.concatenate([ei[1], pad]).reshape(nch, CHUNK)
    xp = jnp.pad(x, ((0, npad - n), (0, 0)))

    deg_call = _make_deg(npad, epad)
    msg_call = _make_msg(npad, epad)
    tc1, tc2, tc3 = _make_tc(npad)

    hists = deg_call(dst)
    h1p = tc1(xp, W1, hists)
    m1 = msg_call(h1p, src, dst)
    h2p = tc2(m1, h1p, hists, W2, b1.reshape(1, D))
    m2 = msg_call(h2p, src, dst)
    outp = tc3(m2, h2p, hists, b2.reshape(1, D))
    return outp[:n]


# speedup vs baseline: 1.0609x; 1.0609x over previous
import functools

import jax
import jax.numpy as jnp
from jax import lax
from jax.experimental import pallas as pl
from jax.experimental.pallas import tpu as pltpu
from jax.experimental.pallas import tpu_sc as plsc

NC = 2
NS = 16
L = 16
D = 128
CHUNK = 128
NBUF = 4

f32 = jnp.float32
i32 = jnp.int32

_SC_PARAMS = dict(
    compiler_params=pltpu.CompilerParams(needs_layout_passes=False))


def _tile_range(wid, nreal):
    flr = nreal // (NC * NS)
    rem = nreal % (NC * NS)
    cnt = flr + jnp.where(wid < rem, 1, 0)
    base = wid * flr + lax.min(wid, rem)
    return base, cnt, flr, rem


def _deg_body(npad, nreal, edge_hbm, out_hbm, hist, dall, sem):
    del sem
    cid = lax.axis_index("c")
    sid = lax.axis_index("s")
    wid = cid * NS + sid
    base, cnt, flr, rem = _tile_range(wid, nreal)
    zeros = jnp.zeros((L,), f32)
    ones = jnp.ones((L,), f32)

    e0 = base * CHUNK
    pltpu.sync_copy(edge_hbm.at[1, pl.ds(e0, flr * CHUNK)],
                    dall.at[pl.ds(0, flr * CHUNK)])

    @pl.when(wid < rem)
    def _():
        pltpu.sync_copy(edge_hbm.at[1, pl.ds(e0 + flr * CHUNK, CHUNK)],
                        dall.at[pl.ds(flr * CHUNK, CHUNK)])

    def zero_body(i, _):
        hist[pl.ds(pl.multiple_of(i * L, L), L)] = zeros
        return 0
    lax.fori_loop(0, npad // L, zero_body, 0)

    def vec_body(j, _):
        idx = dall[pl.ds(pl.multiple_of(j * L, L), L)]
        plsc.addupdate_scatter(hist, [idx], ones)
        return 0
    lax.fori_loop(0, cnt * (CHUNK // L), vec_body, 0)

    pltpu.sync_copy(hist, out_hbm.at[wid])


def _make_deg(npad, nreal):
    flr = nreal // (NC * NS)
    mesh = plsc.VectorSubcoreMesh(core_axis_name="c", subcore_axis_name="s")
    return pl.kernel(
        functools.partial(_deg_body, npad, nreal),
        out_type=jax.ShapeDtypeStruct((NC * NS, npad), f32),
        mesh=mesh,
        scratch_types=[
            pltpu.VMEM((npad,), f32),
            pltpu.VMEM(((flr + 1) * CHUNK,), i32),
            pltpu.SemaphoreType.DMA,
        ],
        **_SC_PARAMS,
    )


def _msg_body(npad, nreal, tab_hbm, edge_hbm, out_hbm,
              acc_sh, rows, sibuf, dibuf,
              gs0, gs1, ss0, ss1, is0, is1, is2, is3, wsem):
    gsems = (gs0, gs1)
    ssems = (ss0, ss1)
    isems = (is0, is1, is2, is3)
    cid = lax.axis_index("c")
    sid = lax.axis_index("s")
    wid = cid * NS + sid
    base, cnt, _, _ = _tile_range(wid, nreal)
    rows_pt = npad // NS
    r0 = sid * rows_pt
    zeros = jnp.zeros((L,), f32)

    def zrow(r, _):
        for cc in range(D // L):
            rows[0, r, pl.ds(cc * L, L)] = zeros
        return 0
    lax.fori_loop(0, CHUNK, zrow, 0)
    nz = rows_pt // CHUNK
    for k in range(nz):
        pltpu.async_copy(rows.at[0], acc_sh.at[pl.ds(r0 + k * CHUNK, CHUNK)],
                         wsem)
    for k in range(nz):
        pltpu.make_async_copy(rows.at[0],
                              acc_sh.at[pl.ds(r0, CHUNK)], wsem).wait()
    plsc.subcore_barrier()

    def fire_idx(c, q):
        e0 = pl.multiple_of((base + c) * CHUNK, CHUNK)
        pltpu.async_copy(edge_hbm.at[0, pl.ds(e0, CHUNK)], sibuf.at[q],
                         isems[q])
        pltpu.async_copy(edge_hbm.at[1, pl.ds(e0, CHUNK)], dibuf.at[q],
                         isems[q])

    def wait_idx(q):
        for _ in range(2):
            pltpu.make_async_copy(edge_hbm.at[0, pl.ds(0, CHUNK)],
                                  sibuf.at[q], isems[q]).wait()

    def fire_g(q, b):
        pltpu.async_copy(tab_hbm.at[sibuf.at[q]], rows.at[b], gsems[b])

    def wait_g(b):
        pltpu.make_async_copy(tab_hbm.at[sibuf.at[0]], rows.at[b],
                              gsems[b]).wait()

    def fire_s(q, b):
        pltpu.async_copy(rows.at[b], acc_sh.at[dibuf.at[q]], ssems[b],
                         add=True)

    def wait_s(b):
        pltpu.make_async_copy(rows.at[b], acc_sh.at[dibuf.at[0]],
                              ssems[b]).wait()

    for q in range(4):
        fire_idx(q, q)
    for c in range(2):
        wait_idx(c)
        fire_g(c, c)

    def iter_body(k, _):
        for cc in range(4):
            c = k * 4 + cc
            b = cc % 2
            q = cc

            @pl.when(c < cnt)
            def _():
                wait_g(b)
                fire_s(q, b)
                wait_s(b)

                @pl.when(c + 4 < cnt)
                def _():
                    fire_idx(c + 4, q)

                @pl.when(c + 2 < cnt)
                def _():
                    wait_idx((q + 2) % 4)
                    fire_g((q + 2) % 4, b)
        return 0
    lax.fori_loop(0, (nreal // (NC * NS) + 1 + 3) // 4, iter_body, 0)
    plsc.subcore_barrier()

    for k in range(nz):
        b = k % 2
        if k >= 2:
            pltpu.make_async_copy(rows.at[0],
                                  out_hbm.at[cid, pl.ds(r0, CHUNK)],
                                  wsem).wait()
        pltpu.sync_copy(acc_sh.at[pl.ds(r0 + k * CHUNK, CHUNK)], rows.at[b])
        pltpu.async_copy(rows.at[b],
                         out_hbm.at[cid, pl.ds(r0 + k * CHUNK, CHUNK)], wsem)
    for k in range(min(nz, 2)):
        pltpu.make_async_copy(rows.at[0], out_hbm.at[cid, pl.ds(r0, CHUNK)],
                              wsem).wait()


def _make_msg(npad, nreal):
    mesh = plsc.VectorSubcoreMesh(core_axis_name="c", subcore_axis_name="s")
    return pl.kernel(
        functools.partial(_msg_body, npad, nreal),
        out_type=jax.ShapeDtypeStruct((NC, npad, D), f32),
        mesh=mesh,
        scratch_types=(
            [
                pltpu.VMEM_SHARED((npad, D), f32),
                pltpu.VMEM((2, CHUNK, D), f32),
                pltpu.VMEM((4, CHUNK), i32),
                pltpu.VMEM((4, CHUNK), i32),
            ]
            + [pltpu.SemaphoreType.DMA] * 9
        ),
        **_SC_PARAMS,
    )


R = 1024


def _dis_from_hist(hb):
    deg = jnp.sum(hb[...], axis=0) + 1.0
    return lax.rsqrt(deg)[:, None]


def _tc1_body(xb, wb, hb, hob):
    h = jnp.dot(xb[...], wb[...], preferred_element_type=f32)
    hob[...] = h * _dis_from_hist(hb)


def _tc2_body(mb, hb, histb, wb, bb, ob):
    dis = _dis_from_hist(histb)
    m = mb[...]
    z = (m[0] + m[1] + hb[...]) * dis + bb[...]
    z = jnp.maximum(z, 0.0)
    ob[...] = jnp.dot(z, wb[...], preferred_element_type=f32) * dis


def _tc3_body(mb, hb, histb, bb, ob):
    dis = _dis_from_hist(histb)
    m = mb[...]
    ob[...] = (m[0] + m[1] + hb[...]) * dis + bb[...]


def _make_tc(npad):
    nb = npad // R
    row = pl.BlockSpec((R, D), lambda i: (i, 0))
    full_w = pl.BlockSpec((D, D), lambda i: (0, 0))
    bias = pl.BlockSpec((1, D), lambda i: (0, 0))
    msg = pl.BlockSpec((NC, R, D), lambda i: (0, i, 0))
    hist = pl.BlockSpec((NC * NS, R), lambda i: (0, i))

    tc1 = pl.pallas_call(
        _tc1_body,
        grid=(nb,),
        in_specs=[row, full_w, hist],
        out_specs=row,
        out_shape=jax.ShapeDtypeStruct((npad, D), f32),
    )
    tc2 = pl.pallas_call(
        _tc2_body,
        grid=(nb,),
        in_specs=[msg, row, hist, full_w, bias],
        out_specs=row,
        out_shape=jax.ShapeDtypeStruct((npad, D), f32),
    )
    tc3 = pl.pallas_call(
        _tc3_body,
        grid=(nb,),
        in_specs=[msg, row, hist, bias],
        out_specs=row,
        out_shape=jax.ShapeDtypeStruct((npad, D), f32),
    )
    return tc1, tc2, tc3


def kernel(x, edge_index, W1, b1, W2, b2):
    n, d = x.shape
    e = edge_index.shape[1]
    assert d == D and e % CHUNK == 0
    nstep = NS * CHUNK
    npad = pl.cdiv(n, nstep) * nstep
    nreal = e // CHUNK

    ei = edge_index.astype(i32)
    xp = jnp.pad(x, ((0, npad - n), (0, 0)))

    deg_call = _make_deg(npad, nreal)
    msg_call = _make_msg(npad, nreal)
    tc1, tc2, tc3 = _make_tc(npad)

    hists = deg_call(ei)
    h1p = tc1(xp, W1, hists)
    m1 = msg_call(h1p, ei)
    h2p = tc2(m1, h1p, hists, W2, b1.reshape(1, D))
    m2 = msg_call(h2p, ei)
    outp = tc3(m2, h2p, hists, b2.reshape(1, D))
    return outp[:n]


# speedup vs baseline: 1.0796x; 1.0176x over previous
import functools

import jax
import jax.numpy as jnp
from jax import lax
from jax.experimental import pallas as pl
from jax.experimental.pallas import tpu as pltpu
from jax.experimental.pallas import tpu_sc as plsc

NC = 2
NS = 16
L = 16
D = 128
CHUNK = 128
NBUF = 4

f32 = jnp.float32
i32 = jnp.int32

_SC_PARAMS = dict(
    compiler_params=pltpu.CompilerParams(needs_layout_passes=False))


def _tile_range(wid, nreal):
    flr = nreal // (NC * NS)
    rem = nreal % (NC * NS)
    cnt = flr + jnp.where(wid < rem, 1, 0)
    base = wid * flr + lax.min(wid, rem)
    return base, cnt, flr, rem


def _deg_body(npad, nreal, edge_hbm, out_hbm, hist, dall, sem):
    del sem
    cid = lax.axis_index("c")
    sid = lax.axis_index("s")
    wid = cid * NS + sid
    base, cnt, flr, rem = _tile_range(wid, nreal)
    zeros = jnp.zeros((L,), f32)
    ones = jnp.ones((L,), f32)

    e0 = base * CHUNK
    pltpu.sync_copy(edge_hbm.at[1, pl.ds(e0, flr * CHUNK)],
                    dall.at[pl.ds(0, flr * CHUNK)])

    @pl.when(wid < rem)
    def _():
        pltpu.sync_copy(edge_hbm.at[1, pl.ds(e0 + flr * CHUNK, CHUNK)],
                        dall.at[pl.ds(flr * CHUNK, CHUNK)])

    def zero_body(i, _):
        hist[pl.ds(pl.multiple_of(i * L, L), L)] = zeros
        return 0
    lax.fori_loop(0, npad // L, zero_body, 0)

    def vec_body(j, _):
        idx = dall[pl.ds(pl.multiple_of(j * L, L), L)]
        plsc.addupdate_scatter(hist, [idx], ones)
        return 0
    lax.fori_loop(0, cnt * (CHUNK // L), vec_body, 0)

    pltpu.sync_copy(hist, out_hbm.at[wid])


def _make_deg(npad, nreal):
    flr = nreal // (NC * NS)
    mesh = plsc.VectorSubcoreMesh(core_axis_name="c", subcore_axis_name="s")
    return pl.kernel(
        functools.partial(_deg_body, npad, nreal),
        out_type=jax.ShapeDtypeStruct((NC * NS, npad), f32),
        mesh=mesh,
        scratch_types=[
            pltpu.VMEM((npad,), f32),
            pltpu.VMEM(((flr + 1) * CHUNK,), i32),
            pltpu.SemaphoreType.DMA,
        ],
        **_SC_PARAMS,
    )


def _msg_body(npad, nreal, tab_hbm, edge_hbm, out_hbm,
              acc_sh, rows, sibuf, dibuf,
              gs0, gs1, ss0, ss1, is0, is1, is2, is3, wsem):
    gsems = (gs0, gs1)
    ssems = (ss0, ss1)
    isems = (is0, is1, is2, is3)
    cid = lax.axis_index("c")
    sid = lax.axis_index("s")
    wid = cid * NS + sid
    base, cnt, _, _ = _tile_range(wid, nreal)
    rows_pt = npad // NS
    r0 = sid * rows_pt
    zeros = jnp.zeros((L,), f32)

    def zrow(r, _):
        for cc in range(D // L):
            rows[0, r, pl.ds(cc * L, L)] = zeros
        return 0
    lax.fori_loop(0, CHUNK, zrow, 0)
    nz = rows_pt // CHUNK
    for k in range(nz):
        pltpu.async_copy(rows.at[0], acc_sh.at[pl.ds(r0 + k * CHUNK, CHUNK)],
                         wsem)
    for k in range(nz):
        pltpu.make_async_copy(rows.at[0],
                              acc_sh.at[pl.ds(r0, CHUNK)], wsem).wait()
    plsc.subcore_barrier()

    def fire_idx(c, q):
        e0 = pl.multiple_of((base + c) * CHUNK, CHUNK)
        pltpu.async_copy(edge_hbm.at[0, pl.ds(e0, CHUNK)], sibuf.at[q],
                         isems[q])
        pltpu.async_copy(edge_hbm.at[1, pl.ds(e0, CHUNK)], dibuf.at[q],
                         isems[q])

    def wait_idx(q):
        for _ in range(2):
            pltpu.make_async_copy(edge_hbm.at[0, pl.ds(0, CHUNK)],
                                  sibuf.at[q], isems[q]).wait()

    def fire_g(q, b):
        pltpu.async_copy(tab_hbm.at[sibuf.at[q]], rows.at[b], gsems[b])

    def wait_g(b):
        pltpu.make_async_copy(tab_hbm.at[sibuf.at[0]], rows.at[b],
                              gsems[b]).wait()

    def fire_s(q, b):
        pltpu.async_copy(rows.at[b], acc_sh.at[dibuf.at[q]], ssems[b],
                         add=True)

    def wait_s(b):
        pltpu.make_async_copy(rows.at[b], acc_sh.at[dibuf.at[0]],
                              ssems[b]).wait()

    for q in range(4):
        fire_idx(q, q)
    for c in range(2):
        wait_idx(c)
        fire_g(c, c)

    def iter_body(k, _):
        for cc in range(4):
            c = k * 4 + cc
            b = cc % 2
            q = cc

            @pl.when(c < cnt)
            def _():
                wait_g(b)
                fire_s(q, b)
                wait_s(b)

                @pl.when(c + 4 < cnt)
                def _():
                    fire_idx(c + 4, q)

                @pl.when(c + 2 < cnt)
                def _():
                    wait_idx((q + 2) % 4)
                    fire_g((q + 2) % 4, b)
        return 0
    lax.fori_loop(0, (nreal // (NC * NS) + 1 + 3) // 4, iter_body, 0)
    plsc.subcore_barrier()

    for k in range(nz):
        b = k % 2
        if k >= 2:
            pltpu.make_async_copy(rows.at[0],
                                  out_hbm.at[cid, pl.ds(r0, CHUNK)],
                                  wsem).wait()
        pltpu.sync_copy(acc_sh.at[pl.ds(r0 + k * CHUNK, CHUNK)], rows.at[b])
        pltpu.async_copy(rows.at[b],
                         out_hbm.at[cid, pl.ds(r0 + k * CHUNK, CHUNK)], wsem)
    for k in range(min(nz, 2)):
        pltpu.make_async_copy(rows.at[0], out_hbm.at[cid, pl.ds(r0, CHUNK)],
                              wsem).wait()


def _make_msg(npad, nreal):
    mesh = plsc.VectorSubcoreMesh(core_axis_name="c", subcore_axis_name="s")
    return pl.kernel(
        functools.partial(_msg_body, npad, nreal),
        out_type=jax.ShapeDtypeStruct((NC, npad, D), f32),
        mesh=mesh,
        scratch_types=(
            [
                pltpu.VMEM_SHARED((npad, D), f32),
                pltpu.VMEM((2, CHUNK, D), f32),
                pltpu.VMEM((4, CHUNK), i32),
                pltpu.VMEM((4, CHUNK), i32),
            ]
            + [pltpu.SemaphoreType.DMA] * 9
        ),
        **_SC_PARAMS,
    )


R = 1024


def _dis_from_hist(hb):
    deg = jnp.sum(hb[...], axis=0) + 1.0
    return lax.rsqrt(deg)[:, None]


def _tc1_body(xb, wb, hb, hob):
    h = jnp.dot(xb[...], wb[...], preferred_element_type=f32)
    hob[...] = h * _dis_from_hist(hb)


def _tc2_body(mb, hb, histb, wb, bb, ob):
    dis = _dis_from_hist(histb)
    m = mb[...]
    z = (m[0] + m[1] + hb[...]) * dis + bb[...]
    z = jnp.maximum(z, 0.0)
    ob[...] = jnp.dot(z, wb[...], preferred_element_type=f32) * dis


def _tc3_body(mb, hb, histb, bb, ob):
    dis = _dis_from_hist(histb)
    m = mb[...]
    ob[...] = (m[0] + m[1] + hb[...]) * dis + bb[...]


def _make_tc(npad, n):
    nb = npad // R
    row = pl.BlockSpec((R, D), lambda i: (i, 0))
    full_w = pl.BlockSpec((D, D), lambda i: (0, 0))
    bias = pl.BlockSpec((1, D), lambda i: (0, 0))
    msg = pl.BlockSpec((NC, R, D), lambda i: (0, i, 0))
    hist = pl.BlockSpec((NC * NS, R), lambda i: (0, i))

    tc1 = pl.pallas_call(
        _tc1_body,
        grid=(nb,),
        in_specs=[row, full_w, hist],
        out_specs=row,
        out_shape=jax.ShapeDtypeStruct((npad, D), f32),
    )
    tc2 = pl.pallas_call(
        _tc2_body,
        grid=(nb,),
        in_specs=[msg, row, hist, full_w, bias],
        out_specs=row,
        out_shape=jax.ShapeDtypeStruct((npad, D), f32),
    )
    tc3 = pl.pallas_call(
        _tc3_body,
        grid=(nb,),
        in_specs=[msg, row, hist, bias],
        out_specs=row,
        out_shape=jax.ShapeDtypeStruct((n, D), f32),
    )
    return tc1, tc2, tc3


def kernel(x, edge_index, W1, b1, W2, b2):
    n, d = x.shape
    e = edge_index.shape[1]
    assert d == D and e % CHUNK == 0
    nstep = NS * CHUNK
    npad = pl.cdiv(n, nstep) * nstep
    nreal = e // CHUNK

    ei = edge_index.astype(i32)

    deg_call = _make_deg(npad, nreal)
    msg_call = _make_msg(npad, nreal)
    tc1, tc2, tc3 = _make_tc(npad, n)

    hists = deg_call(ei)
    h1p = tc1(x, W1, hists)
    m1 = msg_call(h1p, ei)
    h2p = tc2(m1, h1p, hists, W2, b1.reshape(1, D))
    m2 = msg_call(h2p, ei)
    return tc3(m2, h2p, hists, b2.reshape(1, D))


# speedup vs baseline: 1.0997x; 1.0187x over previous
import functools

import jax
import jax.numpy as jnp
from jax import lax
from jax.experimental import pallas as pl
from jax.experimental.pallas import tpu as pltpu
from jax.experimental.pallas import tpu_sc as plsc

NC = 2
NS = 16
L = 16
D = 128
CHUNK = 128
ZROWS = 64

f32 = jnp.float32
i32 = jnp.int32

_SC_PARAMS = dict(
    compiler_params=pltpu.CompilerParams(needs_layout_passes=False))


def _tile_range(wid, nreal):
    flr = nreal // (NC * NS)
    rem = nreal % (NC * NS)
    cnt = flr + jnp.where(wid < rem, 1, 0)
    base = wid * flr + lax.min(wid, rem)
    return base, cnt, flr, rem


def _deg_body(npad, nreal, edge_hbm, out_hbm, hist, dall, sem):
    del sem
    cid = lax.axis_index("c")
    sid = lax.axis_index("s")
    wid = cid * NS + sid
    base, cnt, flr, rem = _tile_range(wid, nreal)
    zeros = jnp.zeros((L,), f32)
    ones = jnp.ones((L,), f32)

    e0 = base * CHUNK
    pltpu.sync_copy(edge_hbm.at[1, pl.ds(e0, flr * CHUNK)],
                    dall.at[pl.ds(0, flr * CHUNK)])

    @pl.when(wid < rem)
    def _():
        pltpu.sync_copy(edge_hbm.at[1, pl.ds(e0 + flr * CHUNK, CHUNK)],
                        dall.at[pl.ds(flr * CHUNK, CHUNK)])

    def zero_body(i, _):
        hist[pl.ds(pl.multiple_of(i * L, L), L)] = zeros
        return 0
    lax.fori_loop(0, npad // L, zero_body, 0)

    def vec_body(j, _):
        idx = dall[pl.ds(pl.multiple_of(j * L, L), L)]
        plsc.addupdate_scatter(hist, [idx], ones)
        return 0
    lax.fori_loop(0, cnt * (CHUNK // L), vec_body, 0)

    pltpu.sync_copy(hist, out_hbm.at[wid])


def _make_deg(npad, nreal):
    flr = nreal // (NC * NS)
    mesh = plsc.VectorSubcoreMesh(core_axis_name="c", subcore_axis_name="s")
    return pl.kernel(
        functools.partial(_deg_body, npad, nreal),
        out_type=jax.ShapeDtypeStruct((NC * NS, npad), f32),
        mesh=mesh,
        scratch_types=[
            pltpu.VMEM((npad,), f32),
            pltpu.VMEM(((flr + 1) * CHUNK,), i32),
            pltpu.SemaphoreType.DMA,
        ],
        **_SC_PARAMS,
    )


def _msg_body(npad, nreal, tab_hbm, edge_hbm, out_hbm,
              acc_sh, rows, zbuf, sibuf, dibuf,
              gs0, gs1, ss0, ss1, is0, is1, is2, is3, wsem):
    gsems = (gs0, gs1)
    ssems = (ss0, ss1)
    isems = (is0, is1, is2, is3)
    cid = lax.axis_index("c")
    sid = lax.axis_index("s")
    wid = cid * NS + sid
    base, cnt, _, _ = _tile_range(wid, nreal)
    rows_pt = npad // NS
    r0 = sid * rows_pt
    zeros = jnp.zeros((L,), f32)
    nz = rows_pt // ZROWS

    def fire_idx(c, q):
        e0 = pl.multiple_of((base + c) * CHUNK, CHUNK)
        pltpu.async_copy(edge_hbm.at[0, pl.ds(e0, CHUNK)], sibuf.at[q],
                         isems[q])
        pltpu.async_copy(edge_hbm.at[1, pl.ds(e0, CHUNK)], dibuf.at[q],
                         isems[q])

    def wait_idx(q):
        for _ in range(2):
            pltpu.make_async_copy(edge_hbm.at[0, pl.ds(0, CHUNK)],
                                  sibuf.at[q], isems[q]).wait()

    def fire_g(q, b):
        pltpu.async_copy(tab_hbm.at[sibuf.at[q]], rows.at[b], gsems[b])

    def wait_g(b):
        pltpu.make_async_copy(tab_hbm.at[sibuf.at[0]], rows.at[b],
                              gsems[b]).wait()

    def fire_s(q, b):
        pltpu.async_copy(rows.at[b], acc_sh.at[dibuf.at[q]], ssems[b],
                         add=True)

    def wait_s(b):
        pltpu.make_async_copy(rows.at[b], acc_sh.at[dibuf.at[0]],
                              ssems[b]).wait()

    for q in range(4):
        fire_idx(q, q)
    for c in range(2):
        wait_idx(c)
        fire_g(c, c)

    def zrow(r, _):
        for cc in range(D // L):
            zbuf[r, pl.ds(cc * L, L)] = zeros
        return 0
    lax.fori_loop(0, ZROWS, zrow, 0)
    for k in range(nz):
        pltpu.async_copy(zbuf, acc_sh.at[pl.ds(r0 + k * ZROWS, ZROWS)], wsem)
    for k in range(nz):
        pltpu.make_async_copy(zbuf, acc_sh.at[pl.ds(r0, ZROWS)], wsem).wait()
    plsc.subcore_barrier()

    def iter_body(k, _):
        for cc in range(4):
            c = k * 4 + cc
            b = cc % 2
            q = cc

            @pl.when(c < cnt)
            def _():
                wait_g(b)
                fire_s(q, b)
                wait_s(b)

                @pl.when(c + 4 < cnt)
                def _():
                    fire_idx(c + 4, q)

                @pl.when(c + 2 < cnt)
                def _():
                    wait_idx((q + 2) % 4)
                    fire_g((q + 2) % 4, b)
        return 0
    lax.fori_loop(0, (nreal // (NC * NS) + 1 + 3) // 4, iter_body, 0)
    plsc.subcore_barrier()

    nco = rows_pt // CHUNK
    for k in range(nco):
        b = k % 2
        if k >= 2:
            pltpu.make_async_copy(rows.at[0],
                                  out_hbm.at[cid, pl.ds(r0, CHUNK)],
                                  wsem).wait()
        pltpu.sync_copy(acc_sh.at[pl.ds(r0 + k * CHUNK, CHUNK)], rows.at[b])
        pltpu.async_copy(rows.at[b],
                         out_hbm.at[cid, pl.ds(r0 + k * CHUNK, CHUNK)], wsem)
    for k in range(min(nco, 2)):
        pltpu.make_async_copy(rows.at[0], out_hbm.at[cid, pl.ds(r0, CHUNK)],
                              wsem).wait()


def _make_msg(npad, nreal):
    mesh = plsc.VectorSubcoreMesh(core_axis_name="c", subcore_axis_name="s")
    return pl.kernel(
        functools.partial(_msg_body, npad, nreal),
        out_type=jax.ShapeDtypeStruct((NC, npad, D), f32),
        mesh=mesh,
        scratch_types=(
            [
                pltpu.VMEM_SHARED((npad, D), f32),
                pltpu.VMEM((2, CHUNK, D), f32),
                pltpu.VMEM((ZROWS, D), f32),
                pltpu.VMEM((4, CHUNK), i32),
                pltpu.VMEM((4, CHUNK), i32),
            ]
            + [pltpu.SemaphoreType.DMA] * 9
        ),
        **_SC_PARAMS,
    )


R = 1024


def _dis_from_hist(hb):
    deg = jnp.sum(hb[...], axis=0) + 1.0
    return lax.rsqrt(deg)[:, None]


def _tc1_body(xb, wb, hb, hob):
    h = jnp.dot(xb[...], wb[...], preferred_element_type=f32)
    hob[...] = h * _dis_from_hist(hb)


def _tc2_body(mb, hb, histb, wb, bb, ob):
    dis = _dis_from_hist(histb)
    m = mb[...]
    z = (m[0] + m[1] + hb[...]) * dis + bb[...]
    z = jnp.maximum(z, 0.0)
    ob[...] = jnp.dot(z, wb[...], preferred_element_type=f32) * dis


def _tc3_body(mb, hb, histb, bb, ob):
    dis = _dis_from_hist(histb)
    m = mb[...]
    ob[...] = (m[0] + m[1] + hb[...]) * dis + bb[...]


def _make_tc(npad, n):
    nb = npad // R
    row = pl.BlockSpec((R, D), lambda i: (i, 0))
    full_w = pl.BlockSpec((D, D), lambda i: (0, 0))
    bias = pl.BlockSpec((1, D), lambda i: (0, 0))
    msg = pl.BlockSpec((NC, R, D), lambda i: (0, i, 0))
    hist = pl.BlockSpec((NC * NS, R), lambda i: (0, i))

    tc1 = pl.pallas_call(
        _tc1_body,
        grid=(nb,),
        in_specs=[row, full_w, hist],
        out_specs=row,
        out_shape=jax.ShapeDtypeStruct((npad, D), f32),
    )
    tc2 = pl.pallas_call(
        _tc2_body,
        grid=(nb,),
        in_specs=[msg, row, hist, full_w, bias],
        out_specs=row,
        out_shape=jax.ShapeDtypeStruct((npad, D), f32),
    )
    tc3 = pl.pallas_call(
        _tc3_body,
        grid=(nb,),
        in_specs=[msg, row, hist, bias],
        out_specs=row,
        out_shape=jax.ShapeDtypeStruct((n, D), f32),
    )
    return tc1, tc2, tc3


def kernel(x, edge_index, W1, b1, W2, b2):
    n, d = x.shape
    e = edge_index.shape[1]
    assert d == D and e % CHUNK == 0
    nstep = NS * CHUNK
    npad = pl.cdiv(n, nstep) * nstep
    nreal = e // CHUNK

    ei = edge_index.astype(i32)

    deg_call = _make_deg(npad, nreal)
    msg_call = _make_msg(npad, nreal)
    tc1, tc2, tc3 = _make_tc(npad, n)

    hists = deg_call(ei)
    h1p = tc1(x, W1, hists)
    m1 = msg_call(h1p, ei)
    h2p = tc2(m1, h1p, hists, W2, b1.reshape(1, D))
    m2 = msg_call(h2p, ei)
    return tc3(m2, h2p, hists, b2.reshape(1, D))


# speedup vs baseline: 1.1353x; 1.0323x over previous
import functools

import jax
import jax.numpy as jnp
from jax import lax
from jax.experimental import pallas as pl
from jax.experimental.pallas import tpu as pltpu
from jax.experimental.pallas import tpu_sc as plsc

NC = 2
NS = 16
L = 16
D = 128
CHUNK = 80
ZROWS = 32

f32 = jnp.float32
i32 = jnp.int32

_SC_PARAMS = dict(
    compiler_params=pltpu.CompilerParams(needs_layout_passes=False))


def _tile_range(wid, nreal):
    flr = nreal // (NC * NS)
    rem = nreal % (NC * NS)
    cnt = flr + jnp.where(wid < rem, 1, 0)
    base = wid * flr + lax.min(wid, rem)
    return base, cnt, flr, rem


def _deg_body(npad, nreal, dst_hbm, out_hbm, hist, dall, sem):
    del sem
    cid = lax.axis_index("c")
    sid = lax.axis_index("s")
    wid = cid * NS + sid
    base, cnt, flr, rem = _tile_range(wid, nreal)
    zeros = jnp.zeros((L,), f32)
    ones = jnp.ones((L,), f32)

    e0 = base * CHUNK
    pltpu.sync_copy(dst_hbm.at[pl.ds(e0, flr * CHUNK)],
                    dall.at[pl.ds(0, flr * CHUNK)])

    @pl.when(wid < rem)
    def _():
        pltpu.sync_copy(dst_hbm.at[pl.ds(e0 + flr * CHUNK, CHUNK)],
                        dall.at[pl.ds(flr * CHUNK, CHUNK)])

    def zero_body(i, _):
        hist[pl.ds(pl.multiple_of(i * L, L), L)] = zeros
        return 0
    lax.fori_loop(0, npad // L, zero_body, 0)

    def vec_body(j, _):
        idx = dall[pl.ds(pl.multiple_of(j * L, L), L)]
        plsc.addupdate_scatter(hist, [idx], ones)
        return 0
    lax.fori_loop(0, cnt * (CHUNK // L), vec_body, 0)

    pltpu.sync_copy(hist, out_hbm.at[wid])


def _make_deg(npad, nreal):
    flr = nreal // (NC * NS)
    mesh = plsc.VectorSubcoreMesh(core_axis_name="c", subcore_axis_name="s")
    return pl.kernel(
        functools.partial(_deg_body, npad, nreal),
        out_type=jax.ShapeDtypeStruct((NC * NS, npad), f32),
        mesh=mesh,
        scratch_types=[
            pltpu.VMEM((npad,), f32),
            pltpu.VMEM(((flr + 1) * CHUNK,), i32),
            pltpu.SemaphoreType.DMA,
        ],
        **_SC_PARAMS,
    )


def _msg_body(npad, nreal, tab_hbm, src_hbm, dst_hbm, out_hbm,
              acc_sh, rows, zbuf, sibuf, dibuf,
              gs0, gs1, gs2, ss0, ss1, ss2, is0, is1, is2, is3, wsem):
    gsems = (gs0, gs1, gs2)
    ssems = (ss0, ss1, ss2)
    isems = (is0, is1, is2, is3)
    cid = lax.axis_index("c")
    sid = lax.axis_index("s")
    wid = cid * NS + sid
    base, cnt, _, _ = _tile_range(wid, nreal)
    rows_pt = npad // NS
    r0 = sid * rows_pt
    zeros = jnp.zeros((L,), f32)
    nz = rows_pt // ZROWS

    def fire_idx(c, q):
        e0 = pl.multiple_of((base + c) * CHUNK, CHUNK)
        pltpu.async_copy(src_hbm.at[pl.ds(e0, CHUNK)], sibuf.at[q],
                         isems[q])
        pltpu.async_copy(dst_hbm.at[pl.ds(e0, CHUNK)], dibuf.at[q],
                         isems[q])

    def wait_idx(q):
        for _ in range(2):
            pltpu.make_async_copy(src_hbm.at[pl.ds(0, CHUNK)],
                                  sibuf.at[q], isems[q]).wait()

    def fire_g(q, b):
        pltpu.async_copy(tab_hbm.at[sibuf.at[q]], rows.at[b], gsems[b])

    def wait_g(b):
        pltpu.make_async_copy(tab_hbm.at[sibuf.at[0]], rows.at[b],
                              gsems[b]).wait()

    def fire_s(q, b):
        pltpu.async_copy(rows.at[b], acc_sh.at[dibuf.at[q]], ssems[b],
                         add=True)

    def wait_s(b):
        pltpu.make_async_copy(rows.at[b], acc_sh.at[dibuf.at[0]],
                              ssems[b]).wait()

    for q in range(4):
        fire_idx(q, q)
    for c in range(3):
        wait_idx(c)
        fire_g(c, c)

    def zrow(r, _):
        for cc in range(D // L):
            zbuf[r, pl.ds(cc * L, L)] = zeros
        return 0
    lax.fori_loop(0, ZROWS, zrow, 0)
    for k in range(nz):
        pltpu.async_copy(zbuf, acc_sh.at[pl.ds(r0 + k * ZROWS, ZROWS)], wsem)
    for k in range(nz):
        pltpu.make_async_copy(zbuf, acc_sh.at[pl.ds(r0, ZROWS)], wsem).wait()
    plsc.subcore_barrier()

    def iter_body(k, _):
        for cc in range(12):
            c = k * 12 + cc
            b = cc % 3
            q = cc % 4

            @pl.when(c < cnt)
            def _():
                wait_g(b)
                fire_s(q, b)
                wait_s(b)

                @pl.when(c + 4 < cnt)
                def _():
                    fire_idx(c + 4, q)

                @pl.when(c + 3 < cnt)
                def _():
                    wait_idx((q + 3) % 4)
                    fire_g((q + 3) % 4, b)
        return 0
    lax.fori_loop(0, (nreal // (NC * NS) + 1 + 11) // 12, iter_body, 0)
    plsc.subcore_barrier()

    nco = rows_pt // CHUNK
    for k in range(nco):
        b = k % 2
        if k >= 2:
            pltpu.make_async_copy(rows.at[0],
                                  out_hbm.at[cid, pl.ds(r0, CHUNK)],
                                  wsem).wait()
        pltpu.sync_copy(acc_sh.at[pl.ds(r0 + k * CHUNK, CHUNK)], rows.at[b])
        pltpu.async_copy(rows.at[b],
                         out_hbm.at[cid, pl.ds(r0 + k * CHUNK, CHUNK)], wsem)
    for k in range(min(nco, 2)):
        pltpu.make_async_copy(rows.at[0], out_hbm.at[cid, pl.ds(r0, CHUNK)],
                              wsem).wait()


def _make_msg(npad, nreal):
    mesh = plsc.VectorSubcoreMesh(core_axis_name="c", subcore_axis_name="s")
    return pl.kernel(
        functools.partial(_msg_body, npad, nreal),
        out_type=jax.ShapeDtypeStruct((NC, npad, D), f32),
        mesh=mesh,
        scratch_types=(
            [
                pltpu.VMEM_SHARED((npad, D), f32),
                pltpu.VMEM((3, CHUNK, D), f32),
                pltpu.VMEM((ZROWS, D), f32),
                pltpu.VMEM((4, CHUNK), i32),
                pltpu.VMEM((4, CHUNK), i32),
            ]
            + [pltpu.SemaphoreType.DMA] * 11
        ),
        **_SC_PARAMS,
    )


R = 1024


def _dis_from_hist(hb):
    deg = jnp.sum(hb[...], axis=0) + 1.0
    return lax.rsqrt(deg)[:, None]


def _tc1_body(xb, wb, hb, hob):
    h = jnp.dot(xb[...], wb[...], preferred_element_type=f32)
    hob[...] = h * _dis_from_hist(hb)


def _tc2_body(mb, hb, histb, wb, bb, ob):
    dis = _dis_from_hist(histb)
    m = mb[...]
    z = (m[0] + m[1] + hb[...]) * dis + bb[...]
    z = jnp.maximum(z, 0.0)
    ob[...] = jnp.dot(z, wb[...], preferred_element_type=f32) * dis


def _tc3_body(mb, hb, histb, bb, ob):
    dis = _dis_from_hist(histb)
    m = mb[...]
    ob[...] = (m[0] + m[1] + hb[...]) * dis + bb[...]


def _make_tc(npad, n):
    nb = npad // R
    row = pl.BlockSpec((R, D), lambda i: (i, 0))
    full_w = pl.BlockSpec((D, D), lambda i: (0, 0))
    bias = pl.BlockSpec((1, D), lambda i: (0, 0))
    msg = pl.BlockSpec((NC, R, D), lambda i: (0, i, 0))
    hist = pl.BlockSpec((NC * NS, R), lambda i: (0, i))

    tc1 = pl.pallas_call(
        _tc1_body,
        grid=(nb,),
        in_specs=[row, full_w, hist],
        out_specs=row,
        out_shape=jax.ShapeDtypeStruct((npad, D), f32),
    )
    tc2 = pl.pallas_call(
        _tc2_body,
        grid=(nb,),
        in_specs=[msg, row, hist, full_w, bias],
        out_specs=row,
        out_shape=jax.ShapeDtypeStruct((npad, D), f32),
    )
    tc3 = pl.pallas_call(
        _tc3_body,
        grid=(nb,),
        in_specs=[msg, row, hist, bias],
        out_specs=row,
        out_shape=jax.ShapeDtypeStruct((n, D), f32),
    )
    return tc1, tc2, tc3


def kernel(x, edge_index, W1, b1, W2, b2):
    n, d = x.shape
    e = edge_index.shape[1]
    assert d == D and e % CHUNK == 0
    nstep = NS * CHUNK
    npad = pl.cdiv(n, nstep) * nstep
    nreal = e // CHUNK

    ei = edge_index.astype(i32)
    src, dst = ei[0], ei[1]

    deg_call = _make_deg(npad, nreal)
    msg_call = _make_msg(npad, nreal)
    tc1, tc2, tc3 = _make_tc(npad, n)

    hists = deg_call(dst)
    h1p = tc1(x, W1, hists)
    m1 = msg_call(h1p, src, dst)
    h2p = tc2(m1, h1p, hists, W2, b1.reshape(1, D))
    m2 = msg_call(h2p, src, dst)
    return tc3(m2, h2p, hists, b2.reshape(1, D))


# speedup vs baseline: 1.1613x; 1.0229x over previous
import functools

import jax
import jax.numpy as jnp
from jax import lax
from jax.experimental import pallas as pl
from jax.experimental.pallas import tpu as pltpu
from jax.experimental.pallas import tpu_sc as plsc

NC = 2
NS = 16
L = 16
D = 128
CHUNK = 64
ZROWS = 32

f32 = jnp.float32
i32 = jnp.int32

_SC_PARAMS = dict(
    compiler_params=pltpu.CompilerParams(needs_layout_passes=False))


def _tile_range(wid, nreal):
    flr = nreal // (NC * NS)
    rem = nreal % (NC * NS)
    cnt = flr + jnp.where(wid < rem, 1, 0)
    base = wid * flr + lax.min(wid, rem)
    return base, cnt, flr, rem


def _deg_body(npad, nreal, dst_hbm, out_hbm, hist, dall, sem):
    del sem
    cid = lax.axis_index("c")
    sid = lax.axis_index("s")
    wid = cid * NS + sid
    base, cnt, flr, rem = _tile_range(wid, nreal)
    zeros = jnp.zeros((L,), f32)
    ones = jnp.ones((L,), f32)

    e0 = base * CHUNK
    pltpu.sync_copy(dst_hbm.at[pl.ds(e0, flr * CHUNK)],
                    dall.at[pl.ds(0, flr * CHUNK)])

    @pl.when(wid < rem)
    def _():
        pltpu.sync_copy(dst_hbm.at[pl.ds(e0 + flr * CHUNK, CHUNK)],
                        dall.at[pl.ds(flr * CHUNK, CHUNK)])

    def zero_body(i, _):
        hist[pl.ds(pl.multiple_of(i * L, L), L)] = zeros
        return 0
    lax.fori_loop(0, npad // L, zero_body, 0)

    def vec_body(j, _):
        idx = dall[pl.ds(pl.multiple_of(j * L, L), L)]
        plsc.addupdate_scatter(hist, [idx], ones)
        return 0
    lax.fori_loop(0, cnt * (CHUNK // L), vec_body, 0)

    pltpu.sync_copy(hist, out_hbm.at[wid])


def _make_deg(npad, nreal):
    flr = nreal // (NC * NS)
    mesh = plsc.VectorSubcoreMesh(core_axis_name="c", subcore_axis_name="s")
    return pl.kernel(
        functools.partial(_deg_body, npad, nreal),
        out_type=jax.ShapeDtypeStruct((NC * NS, npad), f32),
        mesh=mesh,
        scratch_types=[
            pltpu.VMEM((npad,), f32),
            pltpu.VMEM(((flr + 1) * CHUNK,), i32),
            pltpu.SemaphoreType.DMA,
        ],
        **_SC_PARAMS,
    )


def _msg_body(npad, nreal, tab_hbm, src_hbm, dst_hbm, out_hbm,
              acc_sh, rows, zbuf, sibuf, dibuf,
              gs0, gs1, gs2, gs3, ss0, ss1, ss2, ss3,
              is0, is1, is2, is3, is4, is5, is6, is7, wsem):
    gsems = (gs0, gs1, gs2, gs3)
    ssems = (ss0, ss1, ss2, ss3)
    isems = (is0, is1, is2, is3, is4, is5, is6, is7)
    cid = lax.axis_index("c")
    sid = lax.axis_index("s")
    wid = cid * NS + sid
    base, cnt, _, _ = _tile_range(wid, nreal)
    rows_pt = npad // NS
    r0 = sid * rows_pt
    zeros = jnp.zeros((L,), f32)
    nz = rows_pt // ZROWS

    def fire_idx(c, q):
        e0 = pl.multiple_of((base + c) * CHUNK, CHUNK)
        pltpu.async_copy(src_hbm.at[pl.ds(e0, CHUNK)], sibuf.at[q],
                         isems[q])
        pltpu.async_copy(dst_hbm.at[pl.ds(e0, CHUNK)], dibuf.at[q],
                         isems[q])

    def wait_idx(q):
        for _ in range(2):
            pltpu.make_async_copy(src_hbm.at[pl.ds(0, CHUNK)],
                                  sibuf.at[q], isems[q]).wait()

    def fire_g(q, b):
        pltpu.async_copy(tab_hbm.at[sibuf.at[q]], rows.at[b], gsems[b])

    def wait_g(b):
        pltpu.make_async_copy(tab_hbm.at[sibuf.at[0]], rows.at[b],
                              gsems[b]).wait()

    def fire_s(q, b):
        pltpu.async_copy(rows.at[b], acc_sh.at[dibuf.at[q]], ssems[b],
                         add=True)

    def wait_s(b):
        pltpu.make_async_copy(rows.at[b], acc_sh.at[dibuf.at[0]],
                              ssems[b]).wait()

    for q in range(8):
        fire_idx(q, q)
    for c in range(4):
        wait_idx(c)
        fire_g(c, c)

    def zrow(r, _):
        for cc in range(D // L):
            zbuf[r, pl.ds(cc * L, L)] = zeros
        return 0
    lax.fori_loop(0, ZROWS, zrow, 0)
    for k in range(nz):
        pltpu.async_copy(zbuf, acc_sh.at[pl.ds(r0 + k * ZROWS, ZROWS)], wsem)
    for k in range(nz):
        pltpu.make_async_copy(zbuf, acc_sh.at[pl.ds(r0, ZROWS)], wsem).wait()
    plsc.subcore_barrier()

    def iter_body(k, _):
        for cc in range(8):
            c = k * 8 + cc
            b = cc % 4
            q = cc

            @pl.when(c < cnt)
            def _():
                wait_g(b)
                fire_s(q, b)
                wait_s(b)

                @pl.when(c + 8 < cnt)
                def _():
                    fire_idx(c + 8, q)

                @pl.when(c + 4 < cnt)
                def _():
                    wait_idx((q + 4) % 8)
                    fire_g((q + 4) % 8, b)
        return 0
    lax.fori_loop(0, (nreal // (NC * NS) + 1 + 7) // 8, iter_body, 0)
    plsc.subcore_barrier()

    nco = rows_pt // CHUNK
    for k in range(nco):
        b = k % 2
        if k >= 2:
            pltpu.make_async_copy(rows.at[0],
                                  out_hbm.at[cid, pl.ds(r0, CHUNK)],
                                  wsem).wait()
        pltpu.sync_copy(acc_sh.at[pl.ds(r0 + k * CHUNK, CHUNK)], rows.at[b])
        pltpu.async_copy(rows.at[b],
                         out_hbm.at[cid, pl.ds(r0 + k * CHUNK, CHUNK)], wsem)
    for k in range(min(nco, 2)):
        pltpu.make_async_copy(rows.at[0], out_hbm.at[cid, pl.ds(r0, CHUNK)],
                              wsem).wait()


def _make_msg(npad, nreal):
    mesh = plsc.VectorSubcoreMesh(core_axis_name="c", subcore_axis_name="s")
    return pl.kernel(
        functools.partial(_msg_body, npad, nreal),
        out_type=jax.ShapeDtypeStruct((NC, npad, D), f32),
        mesh=mesh,
        scratch_types=(
            [
                pltpu.VMEM_SHARED((npad, D), f32),
                pltpu.VMEM((4, CHUNK, D), f32),
                pltpu.VMEM((ZROWS, D), f32),
                pltpu.VMEM((8, CHUNK), i32),
                pltpu.VMEM((8, CHUNK), i32),
            ]
            + [pltpu.SemaphoreType.DMA] * 17
        ),
        **_SC_PARAMS,
    )


R = 1024


def _dis_from_hist(hb):
    deg = jnp.sum(hb[...], axis=0) + 1.0
    return lax.rsqrt(deg)[:, None]


def _tc1_body(xb, wb, hb, hob):
    h = jnp.dot(xb[...], wb[...], preferred_element_type=f32)
    hob[...] = h * _dis_from_hist(hb)


def _tc2_body(mb, hb, histb, wb, bb, ob):
    dis = _dis_from_hist(histb)
    m = mb[...]
    z = (m[0] + m[1] + hb[...]) * dis + bb[...]
    z = jnp.maximum(z, 0.0)
    ob[...] = jnp.dot(z, wb[...], preferred_element_type=f32) * dis


def _tc3_body(mb, hb, histb, bb, ob):
    dis = _dis_from_hist(histb)
    m = mb[...]
    ob[...] = (m[0] + m[1] + hb[...]) * dis + bb[...]


def _make_tc(npad, n):
    nb = npad // R
    row = pl.BlockSpec((R, D), lambda i: (i, 0))
    full_w = pl.BlockSpec((D, D), lambda i: (0, 0))
    bias = pl.BlockSpec((1, D), lambda i: (0, 0))
    msg = pl.BlockSpec((NC, R, D), lambda i: (0, i, 0))
    hist = pl.BlockSpec((NC * NS, R), lambda i: (0, i))

    tc1 = pl.pallas_call(
        _tc1_body,
        grid=(nb,),
        in_specs=[row, full_w, hist],
        out_specs=row,
        out_shape=jax.ShapeDtypeStruct((npad, D), f32),
    )
    tc2 = pl.pallas_call(
        _tc2_body,
        grid=(nb,),
        in_specs=[msg, row, hist, full_w, bias],
        out_specs=row,
        out_shape=jax.ShapeDtypeStruct((npad, D), f32),
    )
    tc3 = pl.pallas_call(
        _tc3_body,
        grid=(nb,),
        in_specs=[msg, row, hist, bias],
        out_specs=row,
        out_shape=jax.ShapeDtypeStruct((n, D), f32),
    )
    return tc1, tc2, tc3


def kernel(x, edge_index, W1, b1, W2, b2):
    n, d = x.shape
    e = edge_index.shape[1]
    assert d == D and e % CHUNK == 0
    nstep = NS * CHUNK
    npad = pl.cdiv(n, nstep) * nstep
    nreal = e // CHUNK

    ei = edge_index.astype(i32)
    src, dst = ei[0], ei[1]

    deg_call = _make_deg(npad, nreal)
    msg_call = _make_msg(npad, nreal)
    tc1, tc2, tc3 = _make_tc(npad, n)

    hists = deg_call(dst)
    h1p = tc1(x, W1, hists)
    m1 = msg_call(h1p, src, dst)
    h2p = tc2(m1, h1p, hists, W2, b1.reshape(1, D))
    m2 = msg_call(h2p, src, dst)
    return tc3(m2, h2p, hists, b2.reshape(1, D))
